# Initial kernel scaffold; baseline (speedup 1.0000x reference)
import jax
import jax.numpy as jnp
from jax.experimental import pallas as pl


def kernel(node_labels, edge_labels, edge_index, node_emb, edge_emb, W_self0, W_msg0, W_edge0, b0, W_self1, W_msg1, W_edge1, b1, W_self2, W_msg2, W_edge2, b2):
    raise NotImplementedError("write your pallas kernel here")


# speedup vs baseline: 6.3513x; 6.3513x over previous
import functools

import jax
import jax.numpy as jnp
from jax import lax
from jax.experimental import pallas as pl
from jax.experimental.pallas import tpu as pltpu
from jax.experimental.pallas import tpu_sc as plsc

N = 10000
E = 320000
D = 128
ED = 16
NREL = 16
NCLS = 40

NC = 2
NS = 16
NW = NC * NS

CHUNK = 128
NCHUNKS = E // CHUNK
KMAX = -(-NCHUNKS // NW)
RPS = N // NS
SCH = 80
NSC = 400 // SCH
FPS = N * NREL // NS

GCHUNK = 400
NGC = N // GCHUNK

@functools.lru_cache(maxsize=None)
def _mesh():
    return plsc.VectorSubcoreMesh(core_axis_name="c", subcore_axis_name="s",
                                  num_cores=NC, num_subcores=NS)


FCH = 2000


def _embed_hist_body(nlab, elab, edst, nemb, h0_out, cnt_out,
                     nidx, nrows, lidx, didx, fidx, ones_v, stage, cnt_acc,
                     sem):
    c = lax.axis_index("c")
    s = lax.axis_index("s")
    wid = s * NC + c
    zoff = s * FPS
    for j in range(FCH // 16):
        stage[pl.ds(j * 16, 16)] = jnp.zeros((16,), jnp.float32)
    for t in range(FPS // FCH):
        pltpu.sync_copy(stage, cnt_acc.at[pl.ds(zoff + t * FCH, FCH)])
    for j in range(CHUNK // 16):
        ones_v[pl.ds(j * 16, 16)] = jnp.full((16,), 1.0, jnp.float32)

    @pl.when(wid < NGC)
    def _():
        base = wid * GCHUNK
        pltpu.sync_copy(nlab.at[pl.ds(base, GCHUNK)], nidx)
        pltpu.async_copy(nemb.at[nidx], nrows, sem).wait()
        pltpu.sync_copy(nrows, h0_out.at[pl.ds(base, GCHUNK)])

    plsc.subcore_barrier()

    def step(k, carry):
        cid = k * NW + wid

        @pl.when(cid < NCHUNKS)
        def _():
            base = cid * CHUNK
            pltpu.sync_copy(elab.at[pl.ds(base, CHUNK)], lidx)
            pltpu.sync_copy(edst.at[pl.ds(base, CHUNK)], didx)
            for j in range(CHUNK // 16):
                sl = pl.ds(j * 16, 16)
                fidx[sl] = didx[sl] * NREL + lidx[sl]
            pltpu.sync_copy(ones_v, cnt_acc.at[fidx], add=True)

        return carry

    lax.fori_loop(0, KMAX, step, 0)
    plsc.subcore_barrier()
    for t in range(FPS // FCH):
        pltpu.sync_copy(cnt_acc.at[pl.ds(zoff + t * FCH, FCH)], stage)
        pltpu.sync_copy(
            stage, cnt_out.at[pl.ds(c * (N * NREL) + zoff + t * FCH, FCH)])


@functools.lru_cache(maxsize=None)
def _embed_hist_kernel():
    return pl.kernel(
        _embed_hist_body,
        out_type=(jax.ShapeDtypeStruct((N, D), jnp.float32),
                  jax.ShapeDtypeStruct((NC * N * NREL,), jnp.float32)),
        mesh=_mesh(),
        scratch_types=[
            pltpu.VMEM((GCHUNK,), jnp.int32),
            pltpu.VMEM((GCHUNK, D), jnp.float32),
            pltpu.VMEM((CHUNK,), jnp.int32),
            pltpu.VMEM((CHUNK,), jnp.int32),
            pltpu.VMEM((CHUNK,), jnp.int32),
            pltpu.VMEM((CHUNK,), jnp.float32),
            pltpu.VMEM((FCH,), jnp.float32),
            pltpu.VMEM_SHARED((N * NREL,), jnp.float32),
            pltpu.SemaphoreType.DMA,
        ],
    )


def _scatter_body(esrc, edst, h, g_out, sidx, didx, rows, stage, acc, sem):
    c = lax.axis_index("c")
    s = lax.axis_index("s")
    wid = s * NC + c

    def zrow(i, carry):
        for j in range(D // 16):
            stage[i, pl.ds(j * 16, 16)] = jnp.zeros((16,), jnp.float32)
        return carry

    lax.fori_loop(0, SCH, zrow, 0)
    for t in range(NSC):
        pltpu.sync_copy(stage, acc.at[pl.ds(s * GCHUNK + t * SCH, SCH)])

    @pl.when(s + NS < NGC)
    def _():
        for t in range(NSC):
            pltpu.sync_copy(
                stage, acc.at[pl.ds((s + NS) * GCHUNK + t * SCH, SCH)])

    plsc.subcore_barrier()

    def step(k, carry):
        cid = k * NW + wid

        @pl.when(cid < NCHUNKS)
        def _():
            base = cid * CHUNK
            pltpu.sync_copy(esrc.at[pl.ds(base, CHUNK)], sidx)
            pltpu.sync_copy(edst.at[pl.ds(base, CHUNK)], didx)
            pltpu.async_copy(h.at[sidx], rows, sem).wait()
            pltpu.sync_copy(rows, acc.at[didx], add=True)

        return carry

    lax.fori_loop(0, KMAX, step, 0)
    plsc.subcore_barrier()
    for t in range(NSC):
        off = s * GCHUNK + t * SCH
        pltpu.sync_copy(acc.at[pl.ds(off, SCH)], stage)
        pltpu.sync_copy(stage, g_out.at[c, pl.ds(off, SCH)])

    @pl.when(s + NS < NGC)
    def _():
        for t in range(NSC):
            off = (s + NS) * GCHUNK + t * SCH
            pltpu.sync_copy(acc.at[pl.ds(off, SCH)], stage)
            pltpu.sync_copy(stage, g_out.at[c, pl.ds(off, SCH)])


@functools.lru_cache(maxsize=None)
def _scatter_kernel():
    return pl.kernel(
        _scatter_body,
        out_type=jax.ShapeDtypeStruct((NC, N, D), jnp.float32),
        mesh=_mesh(),
        scratch_types=[
            pltpu.VMEM((CHUNK,), jnp.int32),
            pltpu.VMEM((CHUNK,), jnp.int32),
            pltpu.VMEM((CHUNK, D), jnp.float32),
            pltpu.VMEM((SCH, D), jnp.float32),
            pltpu.VMEM_SHARED((N, D), jnp.float32),
            pltpu.SemaphoreType.DMA,
        ],
    )


BN = 2000


def _combine_body(act, h_ref, g0_ref, g1_ref, c0_ref, c1_ref, eemb_ref,
                  ws_ref, wm_ref, we_ref, b_ref, o_ref):
    f32 = jnp.float32
    u = jnp.dot(eemb_ref[...], we_ref[...], preferred_element_type=f32)
    acc = jnp.dot(h_ref[...], ws_ref[...], preferred_element_type=f32)
    acc = acc + jnp.dot(g0_ref[...] + g1_ref[...], wm_ref[...],
                        preferred_element_type=f32)
    acc = acc + jnp.dot(c0_ref[...] + c1_ref[...], u,
                        preferred_element_type=f32)
    acc = acc + b_ref[...]
    if act:
        acc = jnp.maximum(acc, 0.0)
    o_ref[...] = acc


def _combine(h, g0, g1, c0, c1, eemb, ws, wm, we, b, act):
    row = lambda i: (i, 0)
    rep = lambda i: (0, 0)
    return pl.pallas_call(
        functools.partial(_combine_body, act),
        out_shape=jax.ShapeDtypeStruct((N, 128), jnp.float32),
        grid=(N // BN,),
        in_specs=[
            pl.BlockSpec((BN, D), row),
            pl.BlockSpec((BN, D), row),
            pl.BlockSpec((BN, D), row),
            pl.BlockSpec((BN, NREL), row),
            pl.BlockSpec((BN, NREL), row),
            pl.BlockSpec((NREL, ED), rep),
            pl.BlockSpec((D, 128), rep),
            pl.BlockSpec((D, 128), rep),
            pl.BlockSpec((ED, 128), rep),
            pl.BlockSpec((1, 128), rep),
        ],
        out_specs=pl.BlockSpec((BN, 128), row),
    )(h, g0, g1, c0, c1, eemb, ws, wm, we, b)


def kernel(node_labels, edge_labels, edge_index, node_emb, edge_emb,
           W_self0, W_msg0, W_edge0, b0,
           W_self1, W_msg1, W_edge1, b1,
           W_self2, W_msg2, W_edge2, b2):
    i32 = jnp.int32
    f32 = jnp.float32
    nlab = node_labels.astype(i32)
    elab = edge_labels.astype(i32)
    esrc = edge_index[0].astype(i32)
    edst = edge_index[1].astype(i32)
    nemb = node_emb.astype(f32)
    eemb = edge_emb.astype(f32)

    h0, cnt_parts = _embed_hist_kernel()(nlab, elab, edst, nemb)
    cnt_parts = cnt_parts.reshape(NC, N, NREL)
    c0 = cnt_parts[0]
    c1 = cnt_parts[1]

    def padw(w):
        return jnp.pad(w, ((0, 0), (0, 128 - w.shape[1])))

    layers = [
        (W_self0, W_msg0, W_edge0, b0, True),
        (W_self1, W_msg1, W_edge1, b1, True),
        (padw(W_self2), padw(W_msg2), padw(W_edge2),
         jnp.pad(b2, (0, 128 - NCLS)), False),
    ]

    h = h0
    for ws_, wm_, we_, b_, act in layers:
        g = _scatter_kernel()(esrc, edst, h)
        h = _combine(h, g[0], g[1], c0, c1, eemb, ws_, wm_, we_,
                     b_.reshape(1, 128), act)
    return h[:, :NCLS]


# speedup vs baseline: 9.3880x; 1.4781x over previous
import functools

import jax
import jax.numpy as jnp
from jax import lax
from jax.experimental import pallas as pl
from jax.experimental.pallas import tpu as pltpu
from jax.experimental.pallas import tpu_sc as plsc

N = 10000
E = 320000
D = 128
ED = 16
NREL = 16
NCLS = 40

NC = 2
NS = 16
NW = NC * NS

CHUNK = 128
NCHUNKS = E // CHUNK
KMAX = -(-NCHUNKS // NW)
RPS = N // NS
SCH = 80
NSC = 400 // SCH
FPS = N * NREL // NS

GCHUNK = 400
NGC = N // GCHUNK

@functools.lru_cache(maxsize=None)
def _mesh():
    return plsc.VectorSubcoreMesh(core_axis_name="c", subcore_axis_name="s",
                                  num_cores=NC, num_subcores=NS)


FCH = 2000


def _embed_hist_body(nlab, elab, edst, nemb, h0_out, cnt_out,
                     nidx, nrows, lidx, didx, fidx, ones_v, stage, cnt_acc,
                     sem):
    c = lax.axis_index("c")
    s = lax.axis_index("s")
    wid = s * NC + c
    zoff = s * FPS
    for j in range(FCH // 16):
        stage[pl.ds(j * 16, 16)] = jnp.zeros((16,), jnp.float32)
    for t in range(FPS // FCH):
        pltpu.sync_copy(stage, cnt_acc.at[pl.ds(zoff + t * FCH, FCH)])
    for j in range(CHUNK // 16):
        ones_v[pl.ds(j * 16, 16)] = jnp.full((16,), 1.0, jnp.float32)

    @pl.when(wid < NGC)
    def _():
        base = wid * GCHUNK
        pltpu.sync_copy(nlab.at[pl.ds(base, GCHUNK)], nidx)
        pltpu.async_copy(nemb.at[nidx], nrows, sem).wait()
        pltpu.sync_copy(nrows, h0_out.at[pl.ds(base, GCHUNK)])

    plsc.subcore_barrier()

    def step(k, carry):
        cid = k * NW + wid

        @pl.when(cid < NCHUNKS)
        def _():
            base = cid * CHUNK
            pltpu.sync_copy(elab.at[pl.ds(base, CHUNK)], lidx)
            pltpu.sync_copy(edst.at[pl.ds(base, CHUNK)], didx)
            for j in range(CHUNK // 16):
                sl = pl.ds(j * 16, 16)
                fidx[sl] = didx[sl] * NREL + lidx[sl]
            pltpu.sync_copy(ones_v, cnt_acc.at[fidx], add=True)

        return carry

    lax.fori_loop(0, KMAX, step, 0)
    plsc.subcore_barrier()
    for t in range(FPS // FCH):
        pltpu.sync_copy(cnt_acc.at[pl.ds(zoff + t * FCH, FCH)], stage)
        pltpu.sync_copy(
            stage, cnt_out.at[pl.ds(c * (N * NREL) + zoff + t * FCH, FCH)])


@functools.lru_cache(maxsize=None)
def _embed_hist_kernel():
    return pl.kernel(
        _embed_hist_body,
        out_type=(jax.ShapeDtypeStruct((N, D), jnp.float32),
                  jax.ShapeDtypeStruct((NC * N * NREL,), jnp.float32)),
        mesh=_mesh(),
        scratch_types=[
            pltpu.VMEM((GCHUNK,), jnp.int32),
            pltpu.VMEM((GCHUNK, D), jnp.float32),
            pltpu.VMEM((CHUNK,), jnp.int32),
            pltpu.VMEM((CHUNK,), jnp.int32),
            pltpu.VMEM((CHUNK,), jnp.int32),
            pltpu.VMEM((CHUNK,), jnp.float32),
            pltpu.VMEM((FCH,), jnp.float32),
            pltpu.VMEM_SHARED((N * NREL,), jnp.float32),
            pltpu.SemaphoreType.DMA,
        ],
    )


def _scatter_body(esrc, edst, h, g_out, sidx0, sidx1, didx, rows0, rows1,
                  stage, acc, sem0, sem1):
    c = lax.axis_index("c")
    s = lax.axis_index("s")
    wid = s * NC + c

    def start(k, sidx, rows, sem):
        cid = k * NW + wid

        @pl.when(cid < NCHUNKS)
        def _():
            pltpu.sync_copy(esrc.at[pl.ds(cid * CHUNK, CHUNK)], sidx)
            pltpu.async_copy(h.at[sidx], rows, sem)

    def drain_scatter(k, sidx, rows, sem):
        cid = k * NW + wid

        @pl.when(cid < NCHUNKS)
        def _():
            pltpu.sync_copy(edst.at[pl.ds(cid * CHUNK, CHUNK)], didx)
            pltpu.make_async_copy(h.at[sidx], rows, sem).wait()
            pltpu.sync_copy(rows, acc.at[didx], add=True)

    start(0, sidx0, rows0, sem0)

    def zrow(i, carry):
        for j in range(D // 16):
            stage[i, pl.ds(j * 16, 16)] = jnp.zeros((16,), jnp.float32)
        return carry

    lax.fori_loop(0, SCH, zrow, 0)
    for t in range(NSC):
        pltpu.sync_copy(stage, acc.at[pl.ds(s * GCHUNK + t * SCH, SCH)])

    @pl.when(s + NS < NGC)
    def _():
        for t in range(NSC):
            pltpu.sync_copy(
                stage, acc.at[pl.ds((s + NS) * GCHUNK + t * SCH, SCH)])

    plsc.subcore_barrier()

    def step(i, carry):
        k0 = 2 * i
        start(k0 + 1, sidx1, rows1, sem1)
        drain_scatter(k0, sidx0, rows0, sem0)
        start(k0 + 2, sidx0, rows0, sem0)
        drain_scatter(k0 + 1, sidx1, rows1, sem1)
        return carry

    lax.fori_loop(0, (KMAX + 1) // 2, step, 0)
    plsc.subcore_barrier()
    for t in range(NSC):
        off = s * GCHUNK + t * SCH
        pltpu.sync_copy(acc.at[pl.ds(off, SCH)], stage)
        pltpu.sync_copy(stage, g_out.at[c, pl.ds(off, SCH)])

    @pl.when(s + NS < NGC)
    def _():
        for t in range(NSC):
            off = (s + NS) * GCHUNK + t * SCH
            pltpu.sync_copy(acc.at[pl.ds(off, SCH)], stage)
            pltpu.sync_copy(stage, g_out.at[c, pl.ds(off, SCH)])


@functools.lru_cache(maxsize=None)
def _scatter_kernel():
    return pl.kernel(
        _scatter_body,
        out_type=jax.ShapeDtypeStruct((NC, N, D), jnp.float32),
        mesh=_mesh(),
        scratch_types=[
            pltpu.VMEM((CHUNK,), jnp.int32),
            pltpu.VMEM((CHUNK,), jnp.int32),
            pltpu.VMEM((CHUNK,), jnp.int32),
            pltpu.VMEM((CHUNK, D), jnp.float32),
            pltpu.VMEM((CHUNK, D), jnp.float32),
            pltpu.VMEM((SCH, D), jnp.float32),
            pltpu.VMEM_SHARED((N, D), jnp.float32),
            pltpu.SemaphoreType.DMA,
            pltpu.SemaphoreType.DMA,
        ],
    )


BN = 2000


def _combine_body(act, h_ref, g0_ref, g1_ref, c0_ref, c1_ref, eemb_ref,
                  ws_ref, wm_ref, we_ref, b_ref, o_ref):
    f32 = jnp.float32
    u = jnp.dot(eemb_ref[...], we_ref[...], preferred_element_type=f32)
    acc = jnp.dot(h_ref[...], ws_ref[...], preferred_element_type=f32)
    acc = acc + jnp.dot(g0_ref[...] + g1_ref[...], wm_ref[...],
                        preferred_element_type=f32)
    acc = acc + jnp.dot(c0_ref[...] + c1_ref[...], u,
                        preferred_element_type=f32)
    acc = acc + b_ref[...]
    if act:
        acc = jnp.maximum(acc, 0.0)
    o_ref[...] = acc


def _combine(h, g0, g1, c0, c1, eemb, ws, wm, we, b, act):
    row = lambda i: (i, 0)
    rep = lambda i: (0, 0)
    return pl.pallas_call(
        functools.partial(_combine_body, act),
        out_shape=jax.ShapeDtypeStruct((N, 128), jnp.float32),
        grid=(N // BN,),
        in_specs=[
            pl.BlockSpec((BN, D), row),
            pl.BlockSpec((BN, D), row),
            pl.BlockSpec((BN, D), row),
            pl.BlockSpec((BN, NREL), row),
            pl.BlockSpec((BN, NREL), row),
            pl.BlockSpec((NREL, ED), rep),
            pl.BlockSpec((D, 128), rep),
            pl.BlockSpec((D, 128), rep),
            pl.BlockSpec((ED, 128), rep),
            pl.BlockSpec((1, 128), rep),
        ],
        out_specs=pl.BlockSpec((BN, 128), row),
    )(h, g0, g1, c0, c1, eemb, ws, wm, we, b)


def kernel(node_labels, edge_labels, edge_index, node_emb, edge_emb,
           W_self0, W_msg0, W_edge0, b0,
           W_self1, W_msg1, W_edge1, b1,
           W_self2, W_msg2, W_edge2, b2):
    i32 = jnp.int32
    f32 = jnp.float32
    nlab = node_labels.astype(i32)
    elab = edge_labels.astype(i32)
    esrc = edge_index[0].astype(i32)
    edst = edge_index[1].astype(i32)
    nemb = node_emb.astype(f32)
    eemb = edge_emb.astype(f32)

    h0, cnt_parts = _embed_hist_kernel()(nlab, elab, edst, nemb)
    cnt_parts = cnt_parts.reshape(NC, N, NREL)
    c0 = cnt_parts[0]
    c1 = cnt_parts[1]

    def padw(w):
        return jnp.pad(w, ((0, 0), (0, 128 - w.shape[1])))

    layers = [
        (W_self0, W_msg0, W_edge0, b0, True),
        (W_self1, W_msg1, W_edge1, b1, True),
        (padw(W_self2), padw(W_msg2), padw(W_edge2),
         jnp.pad(b2, (0, 128 - NCLS)), False),
    ]

    h = h0
    for ws_, wm_, we_, b_, act in layers:
        g = _scatter_kernel()(esrc, edst, h)
        h = _combine(h, g[0], g[1], c0, c1, eemb, ws_, wm_, we_,
                     b_.reshape(1, 128), act)
    return h[:, :NCLS]
